# initial kernel scaffold (unmeasured)
import jax
import jax.numpy as jnp
from jax import lax
from jax.experimental import pallas as pl
from jax.experimental.pallas import tpu as pltpu

N_DEV = 8

_OFFSETS = (4, 1, 7, 3, 5, 2, 6)


def kernel(x, w_mat, scale_x, scale_w):
    m_full, k_per = x.shape
    k_full, n = w_mat.shape
    m_per = m_full // N_DEV

    def body(x_ref, w_ref, sx_ref, sw_ref, out_ref,
             send_buf, recv_buf, send_sems, recv_sems):
        my = lax.axis_index("i")

        barrier = pltpu.get_barrier_semaphore()
        for d in range(1, N_DEV):
            pl.semaphore_signal(
                barrier, inc=1,
                device_id=(lax.rem(my + d, N_DEV),),
                device_id_type=pl.DeviceIdType.MESH,
            )
        pl.semaphore_wait(barrier, N_DEV - 1)

        sends = []
        for d in _OFFSETS:
            j = lax.rem(my + d, N_DEV)
            send_buf[d] = x_ref[pl.ds(j * m_per, m_per), :].astype(
                jnp.float8_e4m3fn)
            rdma = pltpu.make_async_remote_copy(
                src_ref=send_buf.at[d],
                dst_ref=recv_buf.at[my],
                send_sem=send_sems.at[d],
                recv_sem=recv_sems.at[my],
                device_id=(j,),
                device_id_type=pl.DeviceIdType.MESH,
            )
            rdma.start()
            sends.append(rdma)

        out_ref[...] = jnp.dot(
            x_ref[pl.ds(my * m_per, m_per), :].astype(jnp.bfloat16),
            w_ref[pl.ds(my * k_per, k_per), :].astype(jnp.bfloat16),
            preferred_element_type=jnp.float32,
        )

        for d in _OFFSETS:
            p = lax.rem(my + d, N_DEV)
            recv = pltpu.make_async_remote_copy(
                src_ref=send_buf.at[d],
                dst_ref=recv_buf.at[p],
                send_sem=send_sems.at[d],
                recv_sem=recv_sems.at[p],
                device_id=(p,),
                device_id_type=pl.DeviceIdType.MESH,
            )
            recv.wait_recv()
            out_ref[...] += jnp.dot(
                recv_buf[p].astype(jnp.bfloat16),
                w_ref[pl.ds(p * k_per, k_per), :].astype(jnp.bfloat16),
                preferred_element_type=jnp.float32,
            )

        for rdma in sends:
            rdma.wait_send()

        s = sx_ref[0] * sw_ref[0]
        out_ref[...] = jnp.maximum(out_ref[...] * s, 0.0)

    return pl.pallas_call(
        body,
        out_shape=jax.ShapeDtypeStruct((m_per, n), jnp.float32),
        in_specs=[
            pl.BlockSpec(memory_space=pltpu.VMEM),
            pl.BlockSpec(memory_space=pltpu.VMEM),
            pl.BlockSpec(memory_space=pltpu.SMEM),
            pl.BlockSpec(memory_space=pltpu.SMEM),
        ],
        out_specs=pl.BlockSpec(memory_space=pltpu.VMEM),
        scratch_shapes=[
            pltpu.VMEM((N_DEV, m_per, k_per), jnp.float8_e4m3fn),
            pltpu.VMEM((N_DEV, m_per, k_per), jnp.float8_e4m3fn),
            pltpu.SemaphoreType.DMA((N_DEV,)),
            pltpu.SemaphoreType.DMA((N_DEV,)),
        ],
        compiler_params=pltpu.CompilerParams(collective_id=0),
    )(x, w_mat, scale_x, scale_w)


# baseline (device time: 35109 ns/iter reference)
import jax
import jax.numpy as jnp
from jax import lax
from jax.experimental import pallas as pl
from jax.experimental.pallas import tpu as pltpu

N_DEV = 8

_OFFSETS = (4, 1, 7, 3, 5, 2, 6)


def kernel(x, w_mat, scale_x, scale_w):
    m_full, k_per = x.shape
    k_full, n = w_mat.shape
    m_per = m_full // N_DEV

    def body(x_ref, w_ref, sx_ref, sw_ref, out_ref,
             send_buf, recv_buf, w_buf, send_sems, recv_sems, w_sems):
        my = lax.axis_index("i")

        barrier = pltpu.get_barrier_semaphore()
        for d in range(1, N_DEV):
            pl.semaphore_signal(
                barrier, inc=1,
                device_id=(lax.rem(my + d, N_DEV),),
                device_id_type=pl.DeviceIdType.MESH,
            )
        pl.semaphore_wait(barrier, N_DEV - 1)

        sends = []
        for d in _OFFSETS:
            j = lax.rem(my + d, N_DEV)
            send_buf[d] = x_ref[pl.ds(j * m_per, m_per), :].astype(
                jnp.float8_e4m3fn)
            rdma = pltpu.make_async_remote_copy(
                src_ref=send_buf.at[d],
                dst_ref=recv_buf.at[my],
                send_sem=send_sems.at[d],
                recv_sem=recv_sems.at[my],
                device_id=(j,),
                device_id_type=pl.DeviceIdType.MESH,
            )
            rdma.start()
            sends.append(rdma)

        order = [my] + [lax.rem(my + d, N_DEV) for d in _OFFSETS]

        def w_copy(p, slot):
            return pltpu.make_async_copy(
                w_ref.at[pl.ds(p * k_per, k_per), :],
                w_buf.at[slot],
                w_sems.at[slot],
            )

        cur = w_copy(order[0], 0)
        cur.start()
        for idx in range(N_DEV):
            slot = idx % 2
            nxt = None
            if idx + 1 < N_DEV:
                nxt = w_copy(order[idx + 1], 1 - slot)
                nxt.start()
            cur.wait()
            if idx == 0:
                chunk = x_ref[pl.ds(my * m_per, m_per), :].astype(jnp.bfloat16)
            else:
                p = order[idx]
                recv = pltpu.make_async_remote_copy(
                    src_ref=send_buf.at[0],
                    dst_ref=recv_buf.at[p],
                    send_sem=send_sems.at[0],
                    recv_sem=recv_sems.at[p],
                    device_id=(p,),
                    device_id_type=pl.DeviceIdType.MESH,
                )
                recv.wait_recv()
                chunk = recv_buf[p].astype(jnp.bfloat16)
            partial = jnp.dot(chunk, w_buf[slot].astype(jnp.bfloat16),
                              preferred_element_type=jnp.float32)
            if idx == 0:
                out_ref[...] = partial
            else:
                out_ref[...] += partial
            cur = nxt

        for rdma in sends:
            rdma.wait_send()

        s = sx_ref[0] * sw_ref[0]
        out_ref[...] = jnp.maximum(out_ref[...] * s, 0.0)

    return pl.pallas_call(
        body,
        out_shape=jax.ShapeDtypeStruct((m_per, n), jnp.float32),
        in_specs=[
            pl.BlockSpec(memory_space=pltpu.VMEM),
            pl.BlockSpec(memory_space=pltpu.MemorySpace.HBM),
            pl.BlockSpec(memory_space=pltpu.SMEM),
            pl.BlockSpec(memory_space=pltpu.SMEM),
        ],
        out_specs=pl.BlockSpec(memory_space=pltpu.VMEM),
        scratch_shapes=[
            pltpu.VMEM((N_DEV, m_per, k_per), jnp.float8_e4m3fn),
            pltpu.VMEM((N_DEV, m_per, k_per), jnp.float8_e4m3fn),
            pltpu.VMEM((2, k_per, n), jnp.float32),
            pltpu.SemaphoreType.DMA((N_DEV,)),
            pltpu.SemaphoreType.DMA((N_DEV,)),
            pltpu.SemaphoreType.DMA((2,)),
        ],
        compiler_params=pltpu.CompilerParams(collective_id=0),
    )(x, w_mat, scale_x, scale_w)


# device time: 33821 ns/iter; 1.0381x vs baseline; 1.0381x over previous
import jax
import jax.numpy as jnp
from jax import lax
from jax.experimental import pallas as pl
from jax.experimental.pallas import tpu as pltpu

N_DEV = 8
NBUF = 3


def kernel(x, w_mat, scale_x, scale_w):
    m_full, k_per = x.shape
    k_full, n = w_mat.shape
    m_per = m_full // N_DEV

    def body(x_ref, w_ref, sx_ref, sw_ref, out_ref,
             send_buf, recv_buf, w_buf, send_sems, recv_sems, w_sems):
        my = lax.axis_index("i")
        base = (my // 4) * 4
        r = my - base
        xbase = 4 - base

        def inplane(b, j):
            return b + lax.rem(r + j, 4)

        order = [
            my,
            inplane(base, 1), inplane(base, 3), inplane(base, 2),
            inplane(xbase, 0), inplane(xbase, 1), inplane(xbase, 3),
            inplane(xbase, 2),
        ]
        targets = order[4:] + order[1:4]

        barrier = pltpu.get_barrier_semaphore()
        for d in range(1, N_DEV):
            pl.semaphore_signal(
                barrier, inc=1,
                device_id=(lax.rem(my + d, N_DEV),),
                device_id_type=pl.DeviceIdType.MESH,
            )
        pl.semaphore_wait(barrier, N_DEV - 1)

        sends = []
        for i, j in enumerate(targets):
            send_buf[i] = x_ref[pl.ds(j * m_per, m_per), :].astype(
                jnp.float8_e4m3fn)
            rdma = pltpu.make_async_remote_copy(
                src_ref=send_buf.at[i],
                dst_ref=recv_buf.at[my],
                send_sem=send_sems.at[i],
                recv_sem=recv_sems.at[my],
                device_id=(j,),
                device_id_type=pl.DeviceIdType.MESH,
            )
            rdma.start()
            sends.append(rdma)

        def w_copy(p, slot):
            half = n // 2
            return [
                pltpu.make_async_copy(
                    w_ref.at[pl.ds(p * k_per, k_per), pl.ds(h * half, half)],
                    w_buf.at[slot, slice(None), pl.ds(h * half, half)],
                    w_sems.at[slot, h],
                )
                for h in range(2)
            ]

        copies = {}
        for pre in range(NBUF - 1):
            copies[pre] = w_copy(order[pre], pre)
            for c in copies[pre]:
                c.start()
        for idx in range(N_DEV):
            slot = idx % NBUF
            if idx + NBUF - 1 < N_DEV:
                pid = idx + NBUF - 1
                copies[pid] = w_copy(order[pid], pid % NBUF)
                for c in copies[pid]:
                    c.start()
            for c in copies[idx]:
                c.wait()
            if idx == 0:
                chunk = x_ref[pl.ds(my * m_per, m_per), :].astype(jnp.bfloat16)
            else:
                p = order[idx]
                recv = pltpu.make_async_remote_copy(
                    src_ref=send_buf.at[0],
                    dst_ref=recv_buf.at[p],
                    send_sem=send_sems.at[0],
                    recv_sem=recv_sems.at[p],
                    device_id=(p,),
                    device_id_type=pl.DeviceIdType.MESH,
                )
                recv.wait_recv()
                chunk = recv_buf[p].astype(jnp.bfloat16)
            partial = jnp.dot(chunk, w_buf[slot].astype(jnp.bfloat16),
                              preferred_element_type=jnp.float32)
            if idx == 0:
                out_ref[...] = partial
            else:
                out_ref[...] += partial

        for rdma in sends:
            rdma.wait_send()

        s = sx_ref[0] * sw_ref[0]
        out_ref[...] = jnp.maximum(out_ref[...] * s, 0.0)

    return pl.pallas_call(
        body,
        out_shape=jax.ShapeDtypeStruct((m_per, n), jnp.float32),
        in_specs=[
            pl.BlockSpec(memory_space=pltpu.VMEM),
            pl.BlockSpec(memory_space=pltpu.MemorySpace.HBM),
            pl.BlockSpec(memory_space=pltpu.SMEM),
            pl.BlockSpec(memory_space=pltpu.SMEM),
        ],
        out_specs=pl.BlockSpec(memory_space=pltpu.VMEM),
        scratch_shapes=[
            pltpu.VMEM((N_DEV, m_per, k_per), jnp.float8_e4m3fn),
            pltpu.VMEM((N_DEV, m_per, k_per), jnp.float8_e4m3fn),
            pltpu.VMEM((NBUF, k_per, n), jnp.float32),
            pltpu.SemaphoreType.DMA((N_DEV,)),
            pltpu.SemaphoreType.DMA((N_DEV,)),
            pltpu.SemaphoreType.DMA((NBUF, 2)),
        ],
        compiler_params=pltpu.CompilerParams(collective_id=0),
    )(x, w_mat, scale_x, scale_w)
